# baseline (device time: 271336 ns/iter reference)
import jax
import jax.numpy as jnp
from jax import lax
from jax.experimental import pallas as pl
from jax.experimental.pallas import tpu as pltpu

M = 2048
N = 2048
K = 8192
NX, NY, NZ = 2, 2, 4
N_SLABS = NX * NY
SLAB = M // N_SLABS
CHUNK = SLAB // NZ

BN = 512
BK = 2048

_CompilerParams = getattr(pltpu, "CompilerParams", None) or pltpu.TPUCompilerParams


def _gemm_body(dy_ref, w_ref, out_ref, acc_ref):
    k = pl.program_id(1)

    @pl.when(k == 0)
    def _():
        acc_ref[...] = jnp.zeros_like(acc_ref)

    acc_ref[...] += lax.dot_general(
        dy_ref[...],
        w_ref[...],
        dimension_numbers=(((1,), (1,)), ((), ())),
        preferred_element_type=jnp.float32,
    )

    @pl.when(k == pl.num_programs(1) - 1)
    def _():
        out_ref[...] = acc_ref[...]


def _local_gemm(dy_slab, w):
    return pl.pallas_call(
        _gemm_body,
        grid=(N // BN, K // BK),
        in_specs=[
            pl.BlockSpec((SLAB, BK), lambda n, k: (0, k)),
            pl.BlockSpec((BN, BK), lambda n, k: (n, k)),
        ],
        out_specs=pl.BlockSpec((SLAB, BN), lambda n, k: (0, n)),
        out_shape=jax.ShapeDtypeStruct((SLAB, N), jnp.float32),
        scratch_shapes=[pltpu.VMEM((SLAB, BN), jnp.float32)],
        compiler_params=_CompilerParams(
            dimension_semantics=("parallel", "arbitrary"),
        ),
    )(dy_slab, w)


def _ring_pos(x, y):
    return 2 * x + jnp.bitwise_xor(x, y)


def _coords_of_ring_pos(p):
    x = p // 2
    y = ((p + 1) // 2) % 2
    return x, y


def _slab_of_ring_pos(p):
    x, y = _coords_of_ring_pos(p)
    return 2 * x + y


def _comm_body(partial_ref, out_ref, z_comm, z_ss, z_rs, xy_ss, xy_rs):
    mx = lax.axis_index("x")
    my = lax.axis_index("y")
    mz = lax.axis_index("z")
    s = 2 * mx + my
    r = _ring_pos(mx, my)
    base = s * SLAB

    z_right = (mx, my, (mz + 1) % NZ)
    z_left = (mx, my, (mz - 1) % NZ)
    xr, yr = _coords_of_ring_pos((r + 1) % N_SLABS)
    xl, yl = _coords_of_ring_pos((r - 1) % N_SLABS)
    xy_right = (xr, yr, mz)
    xy_left = (xl, yl, mz)

    bsem = pltpu.get_barrier_semaphore()
    for nbr in (z_left, z_right, xy_left, xy_right):
        pl.semaphore_signal(
            bsem, inc=1, device_id=nbr, device_id_type=pl.DeviceIdType.MESH
        )
    pl.semaphore_wait(bsem, 4)

    out_ref[pl.ds(base, SLAB), :] = partial_ref[...]

    for t in range(NZ - 1):
        c_send = (mz - t) % NZ
        c_recv = (mz - t - 1) % NZ
        rdma = pltpu.make_async_remote_copy(
            src_ref=out_ref.at[pl.ds(base + c_send * CHUNK, CHUNK), :],
            dst_ref=z_comm.at[t],
            send_sem=z_ss.at[t],
            recv_sem=z_rs.at[t],
            device_id=z_right,
            device_id_type=pl.DeviceIdType.MESH,
        )
        rdma.start()
        rdma.wait()
        row = base + c_recv * CHUNK
        out_ref[pl.ds(row, CHUNK), :] = out_ref[pl.ds(row, CHUNK), :] + z_comm[t]

    for h in range(NZ - 1):
        c_send = (mz + 1 - h) % NZ
        row = base + c_send * CHUNK
        rdma = pltpu.make_async_remote_copy(
            src_ref=out_ref.at[pl.ds(row, CHUNK), :],
            dst_ref=out_ref.at[pl.ds(row, CHUNK), :],
            send_sem=z_ss.at[NZ - 1 + h],
            recv_sem=z_rs.at[NZ - 1 + h],
            device_id=z_right,
            device_id_type=pl.DeviceIdType.MESH,
        )
        rdma.start()
        rdma.wait()

    for h in range(N_SLABS - 1):
        p_send = (r - h) % N_SLABS
        row = _slab_of_ring_pos(p_send) * SLAB
        rdma = pltpu.make_async_remote_copy(
            src_ref=out_ref.at[pl.ds(row, SLAB), :],
            dst_ref=out_ref.at[pl.ds(row, SLAB), :],
            send_sem=xy_ss.at[h],
            recv_sem=xy_rs.at[h],
            device_id=xy_right,
            device_id_type=pl.DeviceIdType.MESH,
        )
        rdma.start()
        rdma.wait()


def _allreduce_allgather(partial):
    return pl.pallas_call(
        _comm_body,
        out_shape=jax.ShapeDtypeStruct((M, N), jnp.float32),
        in_specs=[pl.BlockSpec(memory_space=pltpu.VMEM)],
        out_specs=pl.BlockSpec(memory_space=pltpu.VMEM),
        scratch_shapes=[
            pltpu.VMEM((NZ - 1, CHUNK, N), jnp.float32),
            pltpu.SemaphoreType.DMA((2 * (NZ - 1),)),
            pltpu.SemaphoreType.DMA((2 * (NZ - 1),)),
            pltpu.SemaphoreType.DMA((N_SLABS - 1,)),
            pltpu.SemaphoreType.DMA((N_SLABS - 1,)),
        ],
        compiler_params=_CompilerParams(collective_id=0),
    )(partial)


def kernel(dy, W):
    mx = lax.axis_index("x")
    my = lax.axis_index("y")
    s = 2 * mx + my
    dy_slab = lax.dynamic_slice(dy, (s * SLAB, 0), (SLAB, K))
    partial = _local_gemm(dy_slab, W)
    return _allreduce_allgather(partial)


# device time: 204157 ns/iter; 1.3291x vs baseline; 1.3291x over previous
import jax
import jax.numpy as jnp
from jax import lax
from jax.experimental import pallas as pl
from jax.experimental.pallas import tpu as pltpu

M = 2048
N = 2048
K = 8192
NX, NY, NZ = 2, 2, 4
N_SLABS = NX * NY
SLAB = M // N_SLABS
CHUNK = SLAB // NZ

BN = 512
BK = 2048

_CompilerParams = getattr(pltpu, "CompilerParams", None) or pltpu.TPUCompilerParams


def _gemm_body(dy_ref, w_ref, out_ref, acc_ref):
    k = pl.program_id(1)

    @pl.when(k == 0)
    def _():
        acc_ref[...] = jnp.zeros_like(acc_ref)

    acc_ref[...] += lax.dot_general(
        dy_ref[...],
        w_ref[...],
        dimension_numbers=(((1,), (1,)), ((), ())),
        preferred_element_type=jnp.float32,
    )

    @pl.when(k == pl.num_programs(1) - 1)
    def _():
        out_ref[...] = acc_ref[...]


def _local_gemm(dy_slab, w):
    return pl.pallas_call(
        _gemm_body,
        grid=(N // BN, K // BK),
        in_specs=[
            pl.BlockSpec((SLAB, BK), lambda n, k: (0, k)),
            pl.BlockSpec((BN, BK), lambda n, k: (n, k)),
        ],
        out_specs=pl.BlockSpec((SLAB, BN), lambda n, k: (0, n)),
        out_shape=jax.ShapeDtypeStruct((SLAB, N), jnp.float32),
        scratch_shapes=[pltpu.VMEM((SLAB, BN), jnp.float32)],
        compiler_params=_CompilerParams(
            dimension_semantics=("parallel", "arbitrary"),
        ),
    )(dy_slab, w)


def _ring_pos(x, y):
    return 2 * x + jnp.bitwise_xor(x, y)


def _coords_of_ring_pos(p):
    x = p // 2
    y = ((p + 1) // 2) % 2
    return x, y


def _slab_of_ring_pos(p):
    x, y = _coords_of_ring_pos(p)
    return 2 * x + y


def _comm_body(partial_ref, out_ref, z_comm, z_ss, z_rs, xy_ss, xy_rs):
    mx = lax.axis_index("x")
    my = lax.axis_index("y")
    mz = lax.axis_index("z")
    s = 2 * mx + my
    r = _ring_pos(mx, my)
    base = s * SLAB

    z_right = (mx, my, (mz + 1) % NZ)
    z_left = (mx, my, (mz - 1) % NZ)
    xr, yr = _coords_of_ring_pos((r + 1) % N_SLABS)
    xl, yl = _coords_of_ring_pos((r - 1) % N_SLABS)
    xy_right = (xr, yr, mz)
    xy_left = (xl, yl, mz)

    bsem = pltpu.get_barrier_semaphore()
    for nbr in (z_left, z_right, xy_left, xy_right):
        pl.semaphore_signal(
            bsem, inc=1, device_id=nbr, device_id_type=pl.DeviceIdType.MESH
        )
    pl.semaphore_wait(bsem, 4)

    out_ref[pl.ds(base, SLAB), :] = partial_ref[...]

    for t in range(NZ - 1):
        c_send = (mz - t) % NZ
        c_recv = (mz - t - 1) % NZ
        rdma = pltpu.make_async_remote_copy(
            src_ref=out_ref.at[pl.ds(base + c_send * CHUNK, CHUNK), :],
            dst_ref=z_comm.at[t],
            send_sem=z_ss.at[t],
            recv_sem=z_rs.at[t],
            device_id=z_right,
            device_id_type=pl.DeviceIdType.MESH,
        )
        rdma.start()
        rdma.wait()
        row = base + c_recv * CHUNK
        out_ref[pl.ds(row, CHUNK), :] = out_ref[pl.ds(row, CHUNK), :] + z_comm[t]

    for h in range(NZ - 1):
        c_send = (mz + 1 - h) % NZ
        row = base + c_send * CHUNK
        rdma = pltpu.make_async_remote_copy(
            src_ref=out_ref.at[pl.ds(row, CHUNK), :],
            dst_ref=out_ref.at[pl.ds(row, CHUNK), :],
            send_sem=z_ss.at[NZ - 1 + h],
            recv_sem=z_rs.at[NZ - 1 + h],
            device_id=z_right,
            device_id_type=pl.DeviceIdType.MESH,
        )
        rdma.start()
        rdma.wait()

    HALF = SLAB // 2
    for h in range(N_SLABS - 1):
        p_cw = (r - h) % N_SLABS
        p_ccw = (r + h) % N_SLABS
        row_cw = _slab_of_ring_pos(p_cw) * SLAB
        row_ccw = _slab_of_ring_pos(p_ccw) * SLAB + HALF
        cw = pltpu.make_async_remote_copy(
            src_ref=out_ref.at[pl.ds(row_cw, HALF), :],
            dst_ref=out_ref.at[pl.ds(row_cw, HALF), :],
            send_sem=xy_ss.at[2 * h],
            recv_sem=xy_rs.at[2 * h],
            device_id=xy_right,
            device_id_type=pl.DeviceIdType.MESH,
        )
        ccw = pltpu.make_async_remote_copy(
            src_ref=out_ref.at[pl.ds(row_ccw, HALF), :],
            dst_ref=out_ref.at[pl.ds(row_ccw, HALF), :],
            send_sem=xy_ss.at[2 * h + 1],
            recv_sem=xy_rs.at[2 * h + 1],
            device_id=xy_left,
            device_id_type=pl.DeviceIdType.MESH,
        )
        cw.start()
        ccw.start()
        cw.wait()
        ccw.wait()


def _allreduce_allgather(partial):
    return pl.pallas_call(
        _comm_body,
        out_shape=jax.ShapeDtypeStruct((M, N), jnp.float32),
        in_specs=[pl.BlockSpec(memory_space=pltpu.VMEM)],
        out_specs=pl.BlockSpec(memory_space=pltpu.VMEM),
        scratch_shapes=[
            pltpu.VMEM((NZ - 1, CHUNK, N), jnp.float32),
            pltpu.SemaphoreType.DMA((2 * (NZ - 1),)),
            pltpu.SemaphoreType.DMA((2 * (NZ - 1),)),
            pltpu.SemaphoreType.DMA((2 * (N_SLABS - 1),)),
            pltpu.SemaphoreType.DMA((2 * (N_SLABS - 1),)),
        ],
        compiler_params=_CompilerParams(collective_id=0),
    )(partial)


def kernel(dy, W):
    mx = lax.axis_index("x")
    my = lax.axis_index("y")
    s = 2 * mx + my
    dy_slab = lax.dynamic_slice(dy, (s * SLAB, 0), (SLAB, K))
    partial = _local_gemm(dy_slab, W)
    return _allreduce_allgather(partial)


# device time: 165987 ns/iter; 1.6347x vs baseline; 1.2300x over previous
import jax
import jax.numpy as jnp
from jax import lax
from jax.experimental import pallas as pl
from jax.experimental.pallas import tpu as pltpu

M = 2048
N = 2048
K = 8192
NX, NY, NZ = 2, 2, 4
N_SLABS = NX * NY
SLAB = M // N_SLABS
CHUNK = SLAB // NZ

BN = 512
BK = 2048

_CompilerParams = getattr(pltpu, "CompilerParams", None) or pltpu.TPUCompilerParams


def _gemm_body(dy_ref, w_ref, out_ref, acc_ref):
    k = pl.program_id(1)

    @pl.when(k == 0)
    def _():
        acc_ref[...] = jnp.zeros_like(acc_ref)

    acc_ref[...] += lax.dot_general(
        dy_ref[...],
        w_ref[...],
        dimension_numbers=(((1,), (1,)), ((), ())),
        preferred_element_type=jnp.float32,
    )

    @pl.when(k == pl.num_programs(1) - 1)
    def _():
        out_ref[...] = acc_ref[...]


def _local_gemm(dy_slab, w):
    return pl.pallas_call(
        _gemm_body,
        grid=(N // BN, K // BK),
        in_specs=[
            pl.BlockSpec((SLAB, BK), lambda n, k: (0, k)),
            pl.BlockSpec((BN, BK), lambda n, k: (n, k)),
        ],
        out_specs=pl.BlockSpec((SLAB, BN), lambda n, k: (0, n)),
        out_shape=jax.ShapeDtypeStruct((SLAB, N), jnp.float32),
        scratch_shapes=[pltpu.VMEM((SLAB, BN), jnp.float32)],
        compiler_params=_CompilerParams(
            dimension_semantics=("parallel", "arbitrary"),
        ),
    )(dy_slab, w)


def _ring_pos(x, y):
    return 2 * x + jnp.bitwise_xor(x, y)


def _coords_of_ring_pos(p):
    x = p // 2
    y = ((p + 1) // 2) % 2
    return x, y


def _slab_of_ring_pos(p):
    x, y = _coords_of_ring_pos(p)
    return 2 * x + y


def _comm_body(partial_ref, out_ref, z_comm, z_ss, z_rs, xy_ss, xy_rs):
    mx = lax.axis_index("x")
    my = lax.axis_index("y")
    mz = lax.axis_index("z")
    s = 2 * mx + my
    r = _ring_pos(mx, my)
    base = s * SLAB

    z_right = (mx, my, (mz + 1) % NZ)
    z_left = (mx, my, (mz - 1) % NZ)
    xr, yr = _coords_of_ring_pos((r + 1) % N_SLABS)
    xl, yl = _coords_of_ring_pos((r - 1) % N_SLABS)
    xy_right = (xr, yr, mz)
    xy_left = (xl, yl, mz)

    bsem = pltpu.get_barrier_semaphore()
    for nbr in (z_left, z_right, xy_left, xy_right):
        pl.semaphore_signal(
            bsem, inc=1, device_id=nbr, device_id_type=pl.DeviceIdType.MESH
        )
    pl.semaphore_wait(bsem, 4)

    out_ref[pl.ds(base, SLAB), :] = partial_ref[...]

    for t in range(NZ - 1):
        c_send = (mz - t) % NZ
        c_recv = (mz - t - 1) % NZ
        rdma = pltpu.make_async_remote_copy(
            src_ref=out_ref.at[pl.ds(base + c_send * CHUNK, CHUNK), :],
            dst_ref=z_comm.at[t],
            send_sem=z_ss.at[t],
            recv_sem=z_rs.at[t],
            device_id=z_right,
            device_id_type=pl.DeviceIdType.MESH,
        )
        rdma.start()
        rdma.wait()
        row = base + c_recv * CHUNK
        out_ref[pl.ds(row, CHUNK), :] = out_ref[pl.ds(row, CHUNK), :] + z_comm[t]

    c0 = (mz + 1) % NZ
    HC = CHUNK // 2

    def z_hop(g):
        row = base + ((c0 - g) % NZ) * CHUNK
        return pltpu.make_async_remote_copy(
            src_ref=out_ref.at[pl.ds(row, CHUNK), :],
            dst_ref=out_ref.at[pl.ds(row, CHUNK), :],
            send_sem=z_ss.at[NZ - 1 + g],
            recv_sem=z_rs.at[NZ - 1 + g],
            device_id=z_right,
            device_id_type=pl.DeviceIdType.MESH,
        )

    def xy_pair(k, h):
        c = (c0 - k) % NZ
        row_cw = _slab_of_ring_pos((r - h) % N_SLABS) * SLAB + c * CHUNK
        row_ccw = _slab_of_ring_pos((r + h) % N_SLABS) * SLAB + c * CHUNK + HC
        idx = k * 6 + h * 2
        cw = pltpu.make_async_remote_copy(
            src_ref=out_ref.at[pl.ds(row_cw, HC), :],
            dst_ref=out_ref.at[pl.ds(row_cw, HC), :],
            send_sem=xy_ss.at[idx],
            recv_sem=xy_rs.at[idx],
            device_id=xy_right,
            device_id_type=pl.DeviceIdType.MESH,
        )
        ccw = pltpu.make_async_remote_copy(
            src_ref=out_ref.at[pl.ds(row_ccw, HC), :],
            dst_ref=out_ref.at[pl.ds(row_ccw, HC), :],
            send_sem=xy_ss.at[idx + 1],
            recv_sem=xy_rs.at[idx + 1],
            device_id=xy_left,
            device_id_type=pl.DeviceIdType.MESH,
        )
        return cw, ccw

    zh = [z_hop(g) for g in range(NZ - 1)]
    xy = {(k, h): xy_pair(k, h) for k in range(NZ) for h in range(N_SLABS - 1)}

    def start(k, h):
        xy[k, h][0].start()
        xy[k, h][1].start()

    def wait(k, h):
        xy[k, h][0].wait()
        xy[k, h][1].wait()

    zh[0].start()
    start(0, 0)
    wait(0, 0); start(0, 1)
    zh[0].wait()
    zh[1].start()
    start(1, 0)
    wait(0, 1); start(0, 2)
    wait(1, 0); start(1, 1)
    zh[1].wait()
    zh[2].start()
    start(2, 0)
    wait(0, 2)
    wait(1, 1); start(1, 2)
    wait(2, 0); start(2, 1)
    zh[2].wait()
    start(3, 0)
    wait(1, 2)
    wait(2, 1); start(2, 2)
    wait(3, 0); start(3, 1)
    wait(2, 2)
    wait(3, 1); start(3, 2)
    wait(3, 2)


def _allreduce_allgather(partial):
    return pl.pallas_call(
        _comm_body,
        out_shape=jax.ShapeDtypeStruct((M, N), jnp.float32),
        in_specs=[pl.BlockSpec(memory_space=pltpu.VMEM)],
        out_specs=pl.BlockSpec(memory_space=pltpu.VMEM),
        scratch_shapes=[
            pltpu.VMEM((NZ - 1, CHUNK, N), jnp.float32),
            pltpu.SemaphoreType.DMA((2 * (NZ - 1),)),
            pltpu.SemaphoreType.DMA((2 * (NZ - 1),)),
            pltpu.SemaphoreType.DMA((NZ * (N_SLABS - 1) * 2,)),
            pltpu.SemaphoreType.DMA((NZ * (N_SLABS - 1) * 2,)),
        ],
        compiler_params=_CompilerParams(collective_id=0),
    )(partial)


def kernel(dy, W):
    mx = lax.axis_index("x")
    my = lax.axis_index("y")
    s = 2 * mx + my
    dy_slab = lax.dynamic_slice(dy, (s * SLAB, 0), (SLAB, K))
    partial = _local_gemm(dy_slab, W)
    return _allreduce_allgather(partial)
